# baseline (speedup 1.0000x reference)
import functools

import jax
import jax.numpy as jnp
from jax import lax
from jax.experimental import pallas as pl
from jax.experimental.pallas import tpu as pltpu
from jax.experimental.pallas import tpu_sc as plsc

D = 1024
HD = 64
NH = 16
M = 16
DFF = 4096
S = 2048
CAP = 160
MCAP = M * CAP
BUF_ROWS = MCAP + 8
EPS = 1e-9
BR = 256
BQ = 256
BK = 256
NF = 4
FB = DFF // NF
HI = lax.Precision.HIGHEST
F32 = jnp.float32
NEG = -1e30

_SQRT2_INV = 0.7071067811865476

_ERF_T = [+7.853861353153693e-5, -8.010193625184903e-4, +5.188327685732524e-3,
          -2.685381193529856e-2, +1.128358514861418e-1, -3.761262582423300e-1,
          +1.128379165726710e+0]
_ERFC_P = [+2.326819970068386e-2, -1.387039388740657e-1, +3.687424674597105e-1,
           -5.824733027278666e-1, +6.210004621745983e-1, -4.944515323274145e-1,
           +3.404879937665872e-1, -2.741127028184656e-1, +5.638259427386472e-1]
_ERFC_R = [-1.047766399936249e+1, +1.297719955372516e+1, -7.495518717768503e+0,
           +2.921019019210786e+0, -1.015265279202700e+0, +4.218463358204948e-1,
           -2.820767439740514e-1, +5.641895067754075e-1]


def _poly(y, coefs):
    acc = jnp.zeros_like(y)
    for c in coefs:
        acc = acc * y + jnp.float32(c)
    return acc


def _erfc(x):
    ax = jnp.abs(x)
    z = jnp.exp(-x * x)
    q = 1.0 / ax
    y = q * q
    p = jnp.where(ax < 2.0, _poly(y, _ERFC_P), _poly(y, _ERFC_R))
    yv = z * q * p
    yv = jnp.where(-(x * x) < jnp.float32(-88.72283905206835), 0.0, yv)
    big = jnp.where(x < 0.0, 2.0 - yv, yv)
    small = 1.0 - x * _poly(x * x, _ERF_T)
    return jnp.where(ax > 1.0, big, small)


def _gelu_exact(x):
    return 0.5 * x * _erfc(-x * _SQRT2_INV)


def _gelu(x):
    return 0.5 * x * (1.0 + lax.erf(x * _SQRT2_INV))


def _layernorm(xb, g, b):
    mean = jnp.mean(xb, axis=-1, keepdims=True)
    d = xb - mean
    var = jnp.sum(d * d, axis=-1, keepdims=True) / jnp.float32(D - 1)
    return d / jnp.sqrt(var + EPS) * g + b


def _qkv_body(x_ref, w_ref, b_ref, g_ref, bln_ref, o_ref):
    xn = _layernorm(x_ref[...], g_ref[...], bln_ref[...])
    o_ref[...] = (
        jnp.dot(xn, w_ref[...], preferred_element_type=F32)
        + b_ref[...]
    )


def _qkv_call(xf, wqkv, bqkv, g, b):
    return pl.pallas_call(
        _qkv_body,
        grid=(S // BR,),
        in_specs=[
            pl.BlockSpec((BR, D), lambda i: (i, 0)),
            pl.BlockSpec((D, 3 * D), lambda i: (0, 0)),
            pl.BlockSpec((1, 3 * D), lambda i: (0, 0)),
            pl.BlockSpec((1, D), lambda i: (0, 0)),
            pl.BlockSpec((1, D), lambda i: (0, 0)),
        ],
        out_specs=pl.BlockSpec((BR, 3 * D), lambda i: (i, 0)),
        out_shape=jax.ShapeDtypeStruct((S, 3 * D), F32),
    )(xf, wqkv, bqkv, g, b)


def _attn_body(q_ref, kt_ref, v_ref, o_ref, p_ref):
    i = pl.program_id(1)
    q = q_ref[0] * 0.125
    rows = lax.broadcasted_iota(jnp.int32, (BQ, BK), 0) + i * BQ

    def pass1(j, mx):
        off = pl.multiple_of(j * BK, BK)
        s = jnp.dot(q, kt_ref[0, :, pl.ds(off, BK)],
                    preferred_element_type=F32)
        cols = lax.broadcasted_iota(jnp.int32, (BQ, BK), 1) + j * BK
        s = jnp.where(cols > rows, NEG, s)
        p_ref[:, pl.ds(off, BK)] = s
        return jnp.maximum(mx, jnp.max(s, axis=-1, keepdims=True))

    mx = lax.fori_loop(0, i + 1, pass1, jnp.full((BQ, 1), NEG, F32))

    def pass2(j, l):
        off = pl.multiple_of(j * BK, BK)
        p = jnp.exp(p_ref[:, pl.ds(off, BK)] - mx)
        p_ref[:, pl.ds(off, BK)] = p
        return l + jnp.sum(p, axis=-1, keepdims=True)

    l = lax.fori_loop(0, i + 1, pass2, jnp.zeros((BQ, 1), F32))

    def pass3(j, acc):
        off = pl.multiple_of(j * BK, BK)
        a = p_ref[:, pl.ds(off, BK)] / l
        vblk = v_ref[0, pl.ds(off, BK), :]
        return acc + jnp.dot(a, vblk, preferred_element_type=F32)

    o_ref[0] = lax.fori_loop(0, i + 1, pass3, jnp.zeros((BQ, HD), F32))


def _attn_call(q, kt, v):
    return pl.pallas_call(
        _attn_body,
        grid=(NH, S // BQ),
        in_specs=[
            pl.BlockSpec((1, BQ, HD), lambda h, i: (h, i, 0)),
            pl.BlockSpec((1, HD, S), lambda h, i: (h, 0, 0)),
            pl.BlockSpec((1, S, HD), lambda h, i: (h, 0, 0)),
        ],
        out_specs=pl.BlockSpec((1, BQ, HD), lambda h, i: (h, i, 0)),
        out_shape=jax.ShapeDtypeStruct((NH, S, HD), F32),
        scratch_shapes=[pltpu.VMEM((BQ, S), F32)],
    )(q, kt, v)


def _post_body(x_ref, p_ref, w_ref, b_ref, g_ref, bln_ref, x2_ref, h2_ref):
    x2 = (
        x_ref[...]
        + jnp.dot(p_ref[...], w_ref[...], preferred_element_type=F32)
        + b_ref[...]
    )
    x2_ref[...] = x2
    h2_ref[...] = _layernorm(x2, g_ref[...], bln_ref[...])


def _post_call(xf, pre2, wot, bo, g, b):
    return pl.pallas_call(
        _post_body,
        grid=(S // BR,),
        in_specs=[
            pl.BlockSpec((BR, D), lambda i: (i, 0)),
            pl.BlockSpec((BR, D), lambda i: (i, 0)),
            pl.BlockSpec((D, D), lambda i: (0, 0)),
            pl.BlockSpec((1, D), lambda i: (0, 0)),
            pl.BlockSpec((1, D), lambda i: (0, 0)),
            pl.BlockSpec((1, D), lambda i: (0, 0)),
        ],
        out_specs=[
            pl.BlockSpec((BR, D), lambda i: (i, 0)),
            pl.BlockSpec((BR, D), lambda i: (i, 0)),
        ],
        out_shape=[
            jax.ShapeDtypeStruct((S, D), F32),
            jax.ShapeDtypeStruct((S, D), F32),
        ],
    )(xf, pre2, wot, bo, g, b)


def _router_body(h_ref, w1_ref, b1_ref, w2_ref, b2_ref, o_ref):
    hid = (
        jnp.dot(h_ref[...], w1_ref[...], preferred_element_type=F32)
        + b1_ref[...]
    )
    hid = _gelu_exact(hid)
    o_ref[...] = (
        jnp.dot(hid, w2_ref[...], preferred_element_type=F32)
        + b2_ref[...]
    )


def _router_call(h2, w1t, b1, w2p, b2p):
    return pl.pallas_call(
        _router_body,
        grid=(S // BR,),
        in_specs=[
            pl.BlockSpec((BR, D), lambda i: (i, 0)),
            pl.BlockSpec((D, DFF), lambda i: (0, 0)),
            pl.BlockSpec((1, DFF), lambda i: (0, 0)),
            pl.BlockSpec((DFF, 128), lambda i: (0, 0)),
            pl.BlockSpec((1, 128), lambda i: (0, 0)),
        ],
        out_specs=pl.BlockSpec((BR, 128), lambda i: (i, 0)),
        out_shape=jax.ShapeDtypeStruct((S, 128), F32),
    )(h2, w1t, b1, w2p, b2p)


def _meta_body(lg_ref, safe_ref, safec_ref, keep_ref, run_ref):
    i = pl.program_id(0)

    @pl.when(i == 0)
    def _():
        run_ref[...] = jnp.zeros_like(run_ref)

    lg = lg_ref[...]
    lane = lax.broadcasted_iota(jnp.int32, (BR, 128), 1)
    mx = jnp.max(lg, axis=-1, keepdims=True)
    idxv = jnp.min(jnp.where(lg == mx, lane, 128), axis=-1, keepdims=True)
    oh = (lane == idxv).astype(F32)
    r_i = lax.broadcasted_iota(jnp.int32, (BR, BR), 0)
    c_i = lax.broadcasted_iota(jnp.int32, (BR, BR), 1)
    tri = (c_i <= r_i).astype(F32)
    incl = jnp.dot(tri, oh, preferred_element_type=F32) + run_ref[...]
    run_ref[...] = run_ref[...] + jnp.sum(oh, axis=0, keepdims=True)
    pos = jnp.sum(incl * oh, axis=-1, keepdims=True).astype(jnp.int32) - 1
    keep = pos < CAP
    slot = idxv * CAP + pos
    safe_ref[...] = jnp.where(keep, slot, MCAP)
    safec_ref[...] = jnp.where(keep, slot, 0)
    keep_ref[...] = keep.astype(jnp.int32)


def _meta_call(logits_pad):
    return pl.pallas_call(
        _meta_body,
        grid=(S // BR,),
        in_specs=[pl.BlockSpec((BR, 128), lambda i: (i, 0))],
        out_specs=[
            pl.BlockSpec((BR, 1), lambda i: (i, 0)),
            pl.BlockSpec((BR, 1), lambda i: (i, 0)),
            pl.BlockSpec((BR, 1), lambda i: (i, 0)),
        ],
        out_shape=[
            jax.ShapeDtypeStruct((S, 1), jnp.int32),
            jax.ShapeDtypeStruct((S, 1), jnp.int32),
            jax.ShapeDtypeStruct((S, 1), jnp.int32),
        ],
        scratch_shapes=[pltpu.VMEM((1, 128), F32)],
    )(logits_pad)


_NW = 32
_TPW = S // _NW


@functools.cache
def _sc_kernels():
    mesh = plsc.VectorSubcoreMesh(core_axis_name="c", subcore_axis_name="s")

    @functools.partial(
        pl.kernel,
        out_type=jax.ShapeDtypeStruct((BUF_ROWS, D), F32),
        mesh=mesh,
        scratch_types=[
            pltpu.VMEM((_TPW,), jnp.int32),
            pltpu.VMEM((_TPW, D), F32),
            pltpu.SemaphoreType.DMA,
            pltpu.SemaphoreType.DMA,
        ],
    )
    def dispatch(h2_hbm, safe_hbm, buf_hbm, idx_v, rows_v, sem_in, sem_out):
        wid = lax.axis_index("s") * 2 + lax.axis_index("c")
        base = wid * _TPW
        pltpu.sync_copy(safe_hbm.at[pl.ds(base, _TPW)], idx_v)
        pltpu.async_copy(h2_hbm.at[pl.ds(base, _TPW)], rows_v, sem_in).wait()
        pltpu.async_copy(rows_v, buf_hbm.at[idx_v], sem_out).wait()

    @functools.partial(
        pl.kernel,
        out_type=jax.ShapeDtypeStruct((S, D), F32),
        mesh=mesh,
        scratch_types=[
            pltpu.VMEM((_TPW,), jnp.int32),
            pltpu.VMEM((_TPW, D), F32),
            pltpu.SemaphoreType.DMA,
        ],
    )
    def combine(eo_hbm, safec_hbm, gat_hbm, idx_v, rows_v, sem_in):
        wid = lax.axis_index("s") * 2 + lax.axis_index("c")
        base = wid * _TPW
        pltpu.sync_copy(safec_hbm.at[pl.ds(base, _TPW)], idx_v)
        pltpu.async_copy(eo_hbm.at[idx_v], rows_v, sem_in).wait()
        pltpu.sync_copy(rows_v, gat_hbm.at[pl.ds(base, _TPW)])

    return dispatch, combine


def _sc_dispatch(h2, safe):
    return _sc_kernels()[0](h2, safe)


def _sc_combine(eo, safec):
    return _sc_kernels()[1](eo, safec)


def _expert_body(x_ref, w1_ref, w2_ref, o_ref, acc_ref):
    f = pl.program_id(1)
    xb = x_ref[...].astype(jnp.bfloat16)
    h = jnp.dot(xb, w1_ref[0].astype(jnp.bfloat16),
                preferred_element_type=F32)
    h = _gelu(h).astype(jnp.bfloat16)
    part = jnp.dot(h, w2_ref[0].astype(jnp.bfloat16),
                   preferred_element_type=F32)

    @pl.when(f == 0)
    def _():
        acc_ref[...] = part

    @pl.when(f > 0)
    def _():
        acc_ref[...] = acc_ref[...] + part

    @pl.when(f == NF - 1)
    def _():
        o_ref[...] = acc_ref[...]


def _expert_call(buf, mlp1, mlp2):
    return pl.pallas_call(
        _expert_body,
        grid=(M, NF),
        in_specs=[
            pl.BlockSpec((CAP, D), lambda m, f: (m, 0)),
            pl.BlockSpec((1, D, FB), lambda m, f: (m, 0, f)),
            pl.BlockSpec((1, FB, D), lambda m, f: (m, f, 0)),
        ],
        out_specs=pl.BlockSpec((CAP, D), lambda m, f: (m, 0)),
        out_shape=jax.ShapeDtypeStruct((MCAP, D), F32),
        scratch_shapes=[pltpu.VMEM((CAP, D), F32)],
    )(buf, mlp1, mlp2)


def _final_body(x2_ref, gat_ref, h2_ref, keep_ref, o_ref):
    k = keep_ref[...] > 0
    o_ref[...] = x2_ref[...] + jnp.where(k, gat_ref[...], h2_ref[...])


def _final_call(x2, gat, h2, keep):
    return pl.pallas_call(
        _final_body,
        grid=(S // BR,),
        in_specs=[
            pl.BlockSpec((BR, D), lambda i: (i, 0)),
            pl.BlockSpec((BR, D), lambda i: (i, 0)),
            pl.BlockSpec((BR, D), lambda i: (i, 0)),
            pl.BlockSpec((BR, 1), lambda i: (i, 0)),
        ],
        out_specs=pl.BlockSpec((BR, D), lambda i: (i, 0)),
        out_shape=jax.ShapeDtypeStruct((S, D), F32),
    )(x2, gat, h2, keep)


def kernel(x, ln1_b, ln1_g, Wq, bq, Wk, bk, Wv, bv, Wo, bo,
           ln2_b, ln2_g, rw1, rb1, rw2, rb2, mlp1, mlp2):
    xf = x.reshape(S, D)
    wqkv = jnp.concatenate([Wq.T, Wk.T, Wv.T], axis=1)
    bqkv = jnp.concatenate([bq, bk, bv]).reshape(1, 3 * D)
    qkv = _qkv_call(xf, wqkv, bqkv, ln1_g.reshape(1, D), ln1_b.reshape(1, D))

    q = qkv[:, :D].reshape(S, NH, HD).transpose(1, 0, 2)
    kt = qkv[:, D:2 * D].reshape(S, NH, HD).transpose(1, 2, 0)
    v = qkv[:, 2 * D:].reshape(S, NH, HD).transpose(1, 0, 2)
    pre = _attn_call(q, kt, v)
    pre2 = pre.transpose(1, 0, 2).reshape(S, D)

    x2, h2 = _post_call(xf, pre2, Wo.T, bo.reshape(1, D),
                        ln2_g.reshape(1, D), ln2_b.reshape(1, D))

    w2p = jnp.concatenate([rw2.T, jnp.zeros((DFF, 128 - M), F32)], axis=1)
    b2p = jnp.concatenate([rb2, jnp.full((128 - M,), NEG, F32)]).reshape(1, 128)
    logits_pad = _router_call(h2, rw1.T, rb1.reshape(1, DFF), w2p, b2p)

    safe, safec, keep = _meta_call(logits_pad)

    buf = _sc_dispatch(h2, safe.reshape(S))
    eo = _expert_call(buf, mlp1, mlp2)
    gat = _sc_combine(eo, safec.reshape(S))

    out = _final_call(x2, gat, h2, keep)
    logits = logits_pad[:, :M].reshape(1, S, M)
    return out.reshape(1, S, D), logits
